# Initial kernel scaffold; baseline (speedup 1.0000x reference)
import jax
import jax.numpy as jnp
from jax.experimental import pallas as pl


def kernel(x, edge_index, W1, b1, W2, b2):
    raise NotImplementedError("write your pallas kernel here")


# speedup vs baseline: 28.4404x; 28.4404x over previous
import functools

import jax
import jax.numpy as jnp
from jax import lax
from jax.experimental import pallas as pl
from jax.experimental.pallas import tpu as pltpu
from jax.experimental.pallas import tpu_sc as plsc

N = 10000
D = 128
E = 320000
NC = 2
NS = 16
NW = NC * NS
EPW = E // NW
B = 80
NCH = EPW // B
DEG_W = 16

R0 = 624
R_LAST = N - (NS - 1) * R0

_MESH = plsc.VectorSubcoreMesh(core_axis_name="c", subcore_axis_name="s")


def _rows_copy(s, copy_fn):
    @pl.when(s < NS - 1)
    def _():
        copy_fn(s * R0, R0)

    @pl.when(s == NS - 1)
    def _():
        copy_fn((NS - 1) * R0, R_LAST)


def _deg_body(dst_hbm, zeros_hbm, out0_hbm, out1_hbm, acc, dst_v, ones_v):
    c = lax.axis_index("c")
    s = lax.axis_index("s")
    wid = c * NS + s

    @pl.when(s == 0)
    def _():
        pltpu.sync_copy(zeros_hbm, acc)

    pltpu.sync_copy(dst_hbm.at[wid], dst_v)

    @pl.loop(0, B // 16)
    def _(i):
        ones_v[pl.ds(i * 16, 16)] = jnp.ones((16,), jnp.float32)

    plsc.subcore_barrier()

    @pl.loop(0, NCH)
    def _(j):
        pltpu.sync_copy(ones_v, acc.at[dst_v.at[j]], add=True)

    plsc.subcore_barrier()

    @pl.when(jnp.logical_and(c == 0, s == 0))
    def _():
        pltpu.sync_copy(acc, out0_hbm)

    @pl.when(jnp.logical_and(c == 1, s == 0))
    def _():
        pltpu.sync_copy(acc, out1_hbm)


_deg = functools.partial(
    pl.kernel,
    _deg_body,
    out_type=[jax.ShapeDtypeStruct((N,), jnp.float32),
              jax.ShapeDtypeStruct((N,), jnp.float32)],
    mesh=_MESH,
    scratch_types=[
        pltpu.VMEM_SHARED((N,), jnp.float32),
        pltpu.VMEM((NCH, B), jnp.int32),
        pltpu.VMEM((B,), jnp.float32),
    ],
)()


def _agg_body(g_hbm, src_hbm, dst_hbm, out_hbm, acc, src_v, dst_v, buf0, buf1,
              sem0, sem1):
    c = lax.axis_index("c")
    s = lax.axis_index("s")
    wid = c * NS + s
    _rows_copy(s, lambda off, sz: pltpu.sync_copy(
        g_hbm.at[pl.ds(off, sz)], acc.at[pl.ds(off, sz)]))
    pltpu.sync_copy(src_hbm.at[wid], src_v)
    pltpu.sync_copy(dst_hbm.at[wid], dst_v)
    plsc.subcore_barrier()

    def sidx(j):
        return src_v.at[pl.ds(j * B, B)]

    def gwait(buf, sem):
        pltpu.make_async_copy(g_hbm.at[sidx(0)], buf, sem).wait()

    pltpu.async_copy(g_hbm.at[sidx(0)], buf0, sem0)

    @pl.loop(0, NCH // 2)
    def _(k):
        j0 = 2 * k
        pltpu.async_copy(g_hbm.at[sidx(j0 + 1)], buf1, sem1)
        gwait(buf0, sem0)
        pltpu.sync_copy(buf0, acc.at[dst_v.at[j0]], add=True)

        @pl.when(j0 + 2 < NCH)
        def _():
            pltpu.async_copy(g_hbm.at[sidx(j0 + 2)], buf0, sem0)

        gwait(buf1, sem1)
        pltpu.sync_copy(buf1, acc.at[dst_v.at[j0 + 1]], add=True)

    gwait(buf0, sem0)
    pltpu.sync_copy(buf0, acc.at[dst_v.at[NCH - 1]], add=True)

    plsc.subcore_barrier()
    _rows_copy(s, lambda off, sz: pltpu.sync_copy(
        acc.at[pl.ds(off, sz)], out_hbm.at[c, pl.ds(off, sz)]))


_agg = functools.partial(
    pl.kernel,
    _agg_body,
    out_type=jax.ShapeDtypeStruct((NC, N, D), jnp.float32),
    mesh=_MESH,
    scratch_types=[
        pltpu.VMEM_SHARED((N, D), jnp.float32),
        pltpu.VMEM((EPW,), jnp.int32),
        pltpu.VMEM((NCH, B), jnp.int32),
        pltpu.VMEM((B, D), jnp.float32),
        pltpu.VMEM((B, D), jnp.float32),
        pltpu.SemaphoreType.DMA,
        pltpu.SemaphoreType.DMA,
    ],
)()


BM = 512
_GRID = (pl.cdiv(N, BM),)


def _mm(x, W):
    def body(x_ref, w_ref, o_ref):
        o_ref[...] = jnp.dot(x_ref[...], w_ref[...],
                             preferred_element_type=jnp.float32)

    return pl.pallas_call(
        body,
        grid=_GRID,
        in_specs=[pl.BlockSpec((BM, D), lambda i: (i, 0)),
                  pl.BlockSpec((D, D), lambda i: (0, 0))],
        out_specs=pl.BlockSpec((BM, D), lambda i: (i, 0)),
        out_shape=jax.ShapeDtypeStruct((N, D), jnp.float32),
    )(x, W)


_DSPEC = pl.BlockSpec((BM, 1), lambda i: (i, 0))


def _dinv_of(d0, d1):
    return lax.rsqrt(d0 + d1 + 1.0)


def _norm(d0, d1, h1):
    def body(d0_ref, d1_ref, h_ref, o_ref):
        o_ref[...] = h_ref[...] * _dinv_of(d0_ref[...], d1_ref[...])

    return pl.pallas_call(
        body,
        grid=_GRID,
        in_specs=[_DSPEC, _DSPEC,
                  pl.BlockSpec((BM, D), lambda i: (i, 0))],
        out_specs=pl.BlockSpec((BM, D), lambda i: (i, 0)),
        out_shape=jax.ShapeDtypeStruct((N, D), jnp.float32),
    )(d0, d1, h1)


def _post1(d0, d1, p, g1, b1r, W2):
    def body(d0_ref, d1_ref, p_ref, g1_ref, b_ref, w_ref, o_ref):
        dinv = _dinv_of(d0_ref[...], d1_ref[...])
        h = (p_ref[0] + p_ref[1] - g1_ref[...]) * dinv + b_ref[...]
        h = jnp.maximum(h, 0.0)
        o_ref[...] = jnp.dot(h, w_ref[...],
                             preferred_element_type=jnp.float32) * dinv

    return pl.pallas_call(
        body,
        grid=_GRID,
        in_specs=[_DSPEC, _DSPEC,
                  pl.BlockSpec((NC, BM, D), lambda i: (0, i, 0)),
                  pl.BlockSpec((BM, D), lambda i: (i, 0)),
                  pl.BlockSpec((1, D), lambda i: (0, 0)),
                  pl.BlockSpec((D, D), lambda i: (0, 0))],
        out_specs=pl.BlockSpec((BM, D), lambda i: (i, 0)),
        out_shape=jax.ShapeDtypeStruct((N, D), jnp.float32),
    )(d0, d1, p, g1, b1r, W2)


def _post2(d0, d1, q, g2, b2r):
    def body(d0_ref, d1_ref, q_ref, g2_ref, b_ref, o_ref):
        dinv = _dinv_of(d0_ref[...], d1_ref[...])
        o_ref[...] = (q_ref[0] + q_ref[1] - g2_ref[...]) * dinv + b_ref[...]

    return pl.pallas_call(
        body,
        grid=_GRID,
        in_specs=[_DSPEC, _DSPEC,
                  pl.BlockSpec((NC, BM, D), lambda i: (0, i, 0)),
                  pl.BlockSpec((BM, D), lambda i: (i, 0)),
                  pl.BlockSpec((1, D), lambda i: (0, 0))],
        out_specs=pl.BlockSpec((BM, D), lambda i: (i, 0)),
        out_shape=jax.ShapeDtypeStruct((N, D), jnp.float32),
    )(d0, d1, q, g2, b2r)


def kernel(x, edge_index, W1, b1, W2, b2):
    ei = edge_index.astype(jnp.int32)
    src2 = ei[0].reshape(NW, EPW)
    dst3 = ei[1].reshape(NW, NCH, B)
    zeros1 = jnp.zeros((N,), jnp.float32)

    dp0, dp1 = _deg(dst3, zeros1)
    d0, d1 = dp0.reshape(N, 1), dp1.reshape(N, 1)
    h1 = _mm(x, W1)
    g1 = _norm(d0, d1, h1)
    p = _agg(g1, src2, dst3)
    g2 = _post1(d0, d1, p, g1, b1.reshape(1, D), W2)
    q = _agg(g2, src2, dst3)
    return _post2(d0, d1, q, g2, b2.reshape(1, D))


# speedup vs baseline: 35.1616x; 1.2363x over previous
import functools

import jax
import jax.numpy as jnp
from jax import lax
from jax.experimental import pallas as pl
from jax.experimental.pallas import tpu as pltpu
from jax.experimental.pallas import tpu_sc as plsc

N = 10000
D = 128
E = 320000
NC = 2
NS = 16
NW = NC * NS
EPW = E // NW
B = 80
NCH = EPW // B
DEG_W = 16

R0 = 624
R_LAST = N - (NS - 1) * R0

_MESH = plsc.VectorSubcoreMesh(core_axis_name="c", subcore_axis_name="s")


def _rows_copy(s, copy_fn):
    @pl.when(s < NS - 1)
    def _():
        copy_fn(s * R0, R0)

    @pl.when(s == NS - 1)
    def _():
        copy_fn((NS - 1) * R0, R_LAST)


def _deg_body(dst_hbm, zeros_hbm, out0_hbm, out1_hbm, acc, dst_v, ones_v):
    c = lax.axis_index("c")
    s = lax.axis_index("s")
    wid = c * NS + s

    @pl.when(s == 0)
    def _():
        pltpu.sync_copy(zeros_hbm, acc)

    pltpu.sync_copy(dst_hbm.at[wid], dst_v)

    @pl.loop(0, B // 16)
    def _(i):
        ones_v[pl.ds(i * 16, 16)] = jnp.ones((16,), jnp.float32)

    plsc.subcore_barrier()

    @pl.loop(0, NCH)
    def _(j):
        pltpu.sync_copy(ones_v, acc.at[dst_v.at[j]], add=True)

    plsc.subcore_barrier()

    @pl.when(jnp.logical_and(c == 0, s == 0))
    def _():
        pltpu.sync_copy(acc, out0_hbm)

    @pl.when(jnp.logical_and(c == 1, s == 0))
    def _():
        pltpu.sync_copy(acc, out1_hbm)


_deg = functools.partial(
    pl.kernel,
    _deg_body,
    out_type=[jax.ShapeDtypeStruct((N,), jnp.float32),
              jax.ShapeDtypeStruct((N,), jnp.float32)],
    mesh=_MESH,
    scratch_types=[
        pltpu.VMEM_SHARED((N,), jnp.float32),
        pltpu.VMEM((NCH + 3, B), jnp.int32),
        pltpu.VMEM((B,), jnp.float32),
    ],
)()


NGRP = 16
NFULL = NCH // NGRP
NTAIL = NCH - NFULL * NGRP
NCHP = (NFULL + 1) * NGRP
NBUF = 4


def _agg_body(g_hbm, src_hbm, dst_hbm, out_hbm, acc,
              sring0, sring1, dring0, dring1, buf0, buf1, buf2, buf3,
              ssem0, ssem1, dsem0, dsem1, gsem0, gsem1, gsem2, gsem3):
    srings, drings = (sring0, sring1), (dring0, dring1)
    ssems, dsems = (ssem0, ssem1), (dsem0, dsem1)
    bufs, gsems = (buf0, buf1, buf2, buf3), (gsem0, gsem1, gsem2, gsem3)
    c = lax.axis_index("c")
    s = lax.axis_index("s")
    wid = c * NS + s

    def stage(t, p):
        pltpu.async_copy(src_hbm.at[wid, pl.ds(t * NGRP, NGRP)],
                         srings[p], ssems[p])
        pltpu.async_copy(dst_hbm.at[wid, pl.ds(t * NGRP, NGRP)],
                         drings[p], dsems[p])

    def swait(p):
        pltpu.make_async_copy(src_hbm.at[wid, pl.ds(0, NGRP)],
                              srings[p], ssems[p]).wait()

    def dwait(p):
        pltpu.make_async_copy(dst_hbm.at[wid, pl.ds(0, NGRP)],
                              drings[p], dsems[p]).wait()

    def gfire(p, row, b):
        pltpu.async_copy(g_hbm.at[srings[p].at[row]], bufs[b], gsems[b])

    def gwait(b):
        pltpu.make_async_copy(g_hbm.at[srings[0].at[0]],
                              bufs[b], gsems[b]).wait()

    _rows_copy(s, lambda off, sz: pltpu.sync_copy(
        g_hbm.at[pl.ds(off, sz)], acc.at[pl.ds(off, sz)]))

    stage(0, 0)
    swait(0)
    stage(1, 1)
    for b in range(NBUF):
        gfire(0, b, b)
    plsc.subcore_barrier()

    def group_body(p, nch, has_next):
        dwait(p)
        for i in range(nch):
            b = i % NBUF
            gwait(b)
            pltpu.sync_copy(bufs[b], acc.at[drings[p].at[i]], add=True)
            if has_next:
                if i == NGRP - NBUF:
                    swait(1 - p)
                if i < NGRP - NBUF:
                    gfire(p, i + NBUF, b)
                else:
                    gfire(1 - p, i - (NGRP - NBUF), b)
            else:
                if i + NBUF < nch:
                    gfire(p, i + NBUF, b)

    group_body(0, NGRP, True)
    stage(2, 0)

    @pl.loop(0, (NFULL - 1) // 2)
    def _(u):
        t1 = 2 * u + 1
        group_body(1, NGRP, True)
        stage(t1 + 2, 1)
        group_body(0, NGRP, True)

        @pl.when(t1 + 3 <= NFULL)
        def _():
            stage(t1 + 3, 0)

    group_body(NFULL % 2, NTAIL, False)

    plsc.subcore_barrier()
    _rows_copy(s, lambda off, sz: pltpu.sync_copy(
        acc.at[pl.ds(off, sz)], out_hbm.at[c, pl.ds(off, sz)]))


_agg = functools.partial(
    pl.kernel,
    _agg_body,
    out_type=jax.ShapeDtypeStruct((NC, N, D), jnp.float32),
    mesh=_MESH,
    scratch_types=[
        pltpu.VMEM_SHARED((N, D), jnp.float32),
        pltpu.VMEM((NGRP, B), jnp.int32),
        pltpu.VMEM((NGRP, B), jnp.int32),
        pltpu.VMEM((NGRP, B), jnp.int32),
        pltpu.VMEM((NGRP, B), jnp.int32),
        pltpu.VMEM((B, D), jnp.float32),
        pltpu.VMEM((B, D), jnp.float32),
        pltpu.VMEM((B, D), jnp.float32),
        pltpu.VMEM((B, D), jnp.float32),
    ] + [pltpu.SemaphoreType.DMA] * 8,
)()


BM = 1000
_GRID = (pl.cdiv(N, BM),)


def _mm(x, W):
    def body(x_ref, w_ref, o_ref):
        o_ref[...] = jnp.dot(x_ref[...], w_ref[...],
                             preferred_element_type=jnp.float32)

    return pl.pallas_call(
        body,
        grid=_GRID,
        in_specs=[pl.BlockSpec((BM, D), lambda i: (i, 0)),
                  pl.BlockSpec((D, D), lambda i: (0, 0))],
        out_specs=pl.BlockSpec((BM, D), lambda i: (i, 0)),
        out_shape=jax.ShapeDtypeStruct((N, D), jnp.float32),
    )(x, W)


_DSPEC = pl.BlockSpec((BM, 1), lambda i: (i, 0))


def _dinv_of(d0, d1):
    return lax.rsqrt(d0 + d1 + 1.0)


def _norm(d0, d1, h1):
    def body(d0_ref, d1_ref, h_ref, o_ref):
        o_ref[...] = h_ref[...] * _dinv_of(d0_ref[...], d1_ref[...])

    return pl.pallas_call(
        body,
        grid=_GRID,
        in_specs=[_DSPEC, _DSPEC,
                  pl.BlockSpec((BM, D), lambda i: (i, 0))],
        out_specs=pl.BlockSpec((BM, D), lambda i: (i, 0)),
        out_shape=jax.ShapeDtypeStruct((N, D), jnp.float32),
    )(d0, d1, h1)


def _post1(d0, d1, p, g1, b1r, W2):
    def body(d0_ref, d1_ref, p_ref, g1_ref, b_ref, w_ref, o_ref):
        dinv = _dinv_of(d0_ref[...], d1_ref[...])
        h = (p_ref[0] + p_ref[1] - g1_ref[...]) * dinv + b_ref[...]
        h = jnp.maximum(h, 0.0)
        o_ref[...] = jnp.dot(h, w_ref[...],
                             preferred_element_type=jnp.float32) * dinv

    return pl.pallas_call(
        body,
        grid=_GRID,
        in_specs=[_DSPEC, _DSPEC,
                  pl.BlockSpec((NC, BM, D), lambda i: (0, i, 0)),
                  pl.BlockSpec((BM, D), lambda i: (i, 0)),
                  pl.BlockSpec((1, D), lambda i: (0, 0)),
                  pl.BlockSpec((D, D), lambda i: (0, 0))],
        out_specs=pl.BlockSpec((BM, D), lambda i: (i, 0)),
        out_shape=jax.ShapeDtypeStruct((N, D), jnp.float32),
    )(d0, d1, p, g1, b1r, W2)


def _post2(d0, d1, q, g2, b2r):
    def body(d0_ref, d1_ref, q_ref, g2_ref, b_ref, o_ref):
        dinv = _dinv_of(d0_ref[...], d1_ref[...])
        o_ref[...] = (q_ref[0] + q_ref[1] - g2_ref[...]) * dinv + b_ref[...]

    return pl.pallas_call(
        body,
        grid=_GRID,
        in_specs=[_DSPEC, _DSPEC,
                  pl.BlockSpec((NC, BM, D), lambda i: (0, i, 0)),
                  pl.BlockSpec((BM, D), lambda i: (i, 0)),
                  pl.BlockSpec((1, D), lambda i: (0, 0))],
        out_specs=pl.BlockSpec((BM, D), lambda i: (i, 0)),
        out_shape=jax.ShapeDtypeStruct((N, D), jnp.float32),
    )(d0, d1, q, g2, b2r)


def kernel(x, edge_index, W1, b1, W2, b2):
    ei = edge_index.astype(jnp.int32)
    pad = ((0, 0), (0, NCHP - NCH), (0, 0))
    src3 = jnp.pad(ei[0].reshape(NW, NCH, B), pad)
    dst3 = jnp.pad(ei[1].reshape(NW, NCH, B), pad)
    zeros1 = jnp.zeros((N,), jnp.float32)

    dp0, dp1 = _deg(dst3, zeros1)
    d0, d1 = dp0.reshape(N, 1), dp1.reshape(N, 1)
    h1 = _mm(x, W1)
    g1 = _norm(d0, d1, h1)
    p = _agg(g1, src3, dst3)
    g2 = _post1(d0, d1, p, g1, b1.reshape(1, D), W2)
    q = _agg(g2, src3, dst3)
    return _post2(d0, d1, q, g2, b2.reshape(1, D))


# speedup vs baseline: 36.1477x; 1.0280x over previous
import functools

import jax
import jax.numpy as jnp
from jax import lax
from jax.experimental import pallas as pl
from jax.experimental.pallas import tpu as pltpu
from jax.experimental.pallas import tpu_sc as plsc

N = 10000
D = 128
E = 320000
NC = 2
NS = 16
NW = NC * NS
EPW = E // NW
B = 80
NCH = EPW // B
DEG_W = 16

R0 = 624
R_LAST = N - (NS - 1) * R0

_MESH = plsc.VectorSubcoreMesh(core_axis_name="c", subcore_axis_name="s")


def _rows_copy(s, copy_fn):
    @pl.when(s < NS - 1)
    def _():
        copy_fn(s * R0, R0)

    @pl.when(s == NS - 1)
    def _():
        copy_fn((NS - 1) * R0, R_LAST)


def _deg_body(dst_hbm, zeros_hbm, out0_hbm, out1_hbm, acc, dst_v, ones_v,
              *sems):
    c = lax.axis_index("c")
    s = lax.axis_index("s")
    wid = c * NS + s

    @pl.when(s == 0)
    def _():
        pltpu.sync_copy(zeros_hbm, acc)

    pltpu.sync_copy(dst_hbm.at[wid], dst_v)

    @pl.loop(0, B // 16)
    def _(i):
        ones_v[pl.ds(i * 16, 16)] = jnp.ones((16,), jnp.float32)

    plsc.subcore_barrier()

    ndeep = len(sems)

    def fire(j, b):
        pltpu.async_copy(ones_v, acc.at[dst_v.at[j]], sems[b], add=True)

    def drain(b):
        pltpu.make_async_copy(ones_v, acc.at[dst_v.at[0]], sems[b]).wait()

    for b in range(ndeep):
        fire(b, b)

    @pl.loop(0, NCH // ndeep - 1)
    def _(t):
        j0 = (t + 1) * ndeep
        for b in range(ndeep):
            drain(b)
            fire(j0 + b, b)

    for r in range((NCH // ndeep) * ndeep, NCH):
        drain(r % ndeep)
        fire(r, r % ndeep)
    for r in range(NCH - ndeep, NCH):
        drain(r % ndeep)

    plsc.subcore_barrier()

    @pl.when(jnp.logical_and(c == 0, s == 0))
    def _():
        pltpu.sync_copy(acc, out0_hbm)

    @pl.when(jnp.logical_and(c == 1, s == 0))
    def _():
        pltpu.sync_copy(acc, out1_hbm)


_deg = functools.partial(
    pl.kernel,
    _deg_body,
    out_type=[jax.ShapeDtypeStruct((N,), jnp.float32),
              jax.ShapeDtypeStruct((N,), jnp.float32)],
    mesh=_MESH,
    scratch_types=[
        pltpu.VMEM_SHARED((N,), jnp.float32),
        pltpu.VMEM((NCH + 3, B), jnp.int32),
        pltpu.VMEM((B,), jnp.float32),
    ] + [pltpu.SemaphoreType.DMA] * 8,
)()


NGRP = 16
NFULL = NCH // NGRP
NTAIL = NCH - NFULL * NGRP
NCHP = (NFULL + 1) * NGRP
NBUF = 4


def _agg_body(g_hbm, src_hbm, dst_hbm, out_hbm, acc,
              sring0, sring1, dring0, dring1, buf0, buf1, buf2, buf3,
              ssem0, ssem1, dsem0, dsem1, gsem0, gsem1, gsem2, gsem3):
    srings, drings = (sring0, sring1), (dring0, dring1)
    ssems, dsems = (ssem0, ssem1), (dsem0, dsem1)
    bufs, gsems = (buf0, buf1, buf2, buf3), (gsem0, gsem1, gsem2, gsem3)
    c = lax.axis_index("c")
    s = lax.axis_index("s")
    wid = c * NS + s

    def stage(t, p):
        pltpu.async_copy(src_hbm.at[wid, pl.ds(t * NGRP, NGRP)],
                         srings[p], ssems[p])
        pltpu.async_copy(dst_hbm.at[wid, pl.ds(t * NGRP, NGRP)],
                         drings[p], dsems[p])

    def swait(p):
        pltpu.make_async_copy(src_hbm.at[wid, pl.ds(0, NGRP)],
                              srings[p], ssems[p]).wait()

    def dwait(p):
        pltpu.make_async_copy(dst_hbm.at[wid, pl.ds(0, NGRP)],
                              drings[p], dsems[p]).wait()

    def gfire(p, row, b):
        pltpu.async_copy(g_hbm.at[srings[p].at[row]], bufs[b], gsems[b])

    def gwait(b):
        pltpu.make_async_copy(g_hbm.at[srings[0].at[0]],
                              bufs[b], gsems[b]).wait()

    _rows_copy(s, lambda off, sz: pltpu.sync_copy(
        g_hbm.at[pl.ds(off, sz)], acc.at[pl.ds(off, sz)]))

    stage(0, 0)
    swait(0)
    stage(1, 1)
    for b in range(NBUF):
        gfire(0, b, b)
    plsc.subcore_barrier()

    def group_body(p, nch, has_next):
        dwait(p)
        for i in range(nch):
            b = i % NBUF
            gwait(b)
            pltpu.sync_copy(bufs[b], acc.at[drings[p].at[i]], add=True)
            if has_next:
                if i == NGRP - NBUF:
                    swait(1 - p)
                if i < NGRP - NBUF:
                    gfire(p, i + NBUF, b)
                else:
                    gfire(1 - p, i - (NGRP - NBUF), b)
            else:
                if i + NBUF < nch:
                    gfire(p, i + NBUF, b)

    group_body(0, NGRP, True)
    stage(2, 0)

    @pl.loop(0, (NFULL - 1) // 2)
    def _(u):
        t1 = 2 * u + 1
        group_body(1, NGRP, True)
        stage(t1 + 2, 1)
        group_body(0, NGRP, True)

        @pl.when(t1 + 3 <= NFULL)
        def _():
            stage(t1 + 3, 0)

    group_body(NFULL % 2, NTAIL, False)

    plsc.subcore_barrier()
    _rows_copy(s, lambda off, sz: pltpu.sync_copy(
        acc.at[pl.ds(off, sz)], out_hbm.at[c, pl.ds(off, sz)]))


_agg = functools.partial(
    pl.kernel,
    _agg_body,
    out_type=jax.ShapeDtypeStruct((NC, N, D), jnp.float32),
    mesh=_MESH,
    scratch_types=[
        pltpu.VMEM_SHARED((N, D), jnp.float32),
        pltpu.VMEM((NGRP, B), jnp.int32),
        pltpu.VMEM((NGRP, B), jnp.int32),
        pltpu.VMEM((NGRP, B), jnp.int32),
        pltpu.VMEM((NGRP, B), jnp.int32),
        pltpu.VMEM((B, D), jnp.float32),
        pltpu.VMEM((B, D), jnp.float32),
        pltpu.VMEM((B, D), jnp.float32),
        pltpu.VMEM((B, D), jnp.float32),
    ] + [pltpu.SemaphoreType.DMA] * 8,
)()


BM = 1000
_GRID = (pl.cdiv(N, BM),)


_DSPEC = pl.BlockSpec((BM, 1), lambda i: (i, 0))


def _dinv_of(d0, d1):
    return lax.rsqrt(d0 + d1 + 1.0)


def _mmnorm(d0, d1, x, W):
    def body(d0_ref, d1_ref, x_ref, w_ref, o_ref):
        o_ref[...] = jnp.dot(x_ref[...], w_ref[...],
                             preferred_element_type=jnp.float32
                             ) * _dinv_of(d0_ref[...], d1_ref[...])

    return pl.pallas_call(
        body,
        grid=_GRID,
        in_specs=[_DSPEC, _DSPEC,
                  pl.BlockSpec((BM, D), lambda i: (i, 0)),
                  pl.BlockSpec((D, D), lambda i: (0, 0))],
        out_specs=pl.BlockSpec((BM, D), lambda i: (i, 0)),
        out_shape=jax.ShapeDtypeStruct((N, D), jnp.float32),
    )(d0, d1, x, W)


def _post1(d0, d1, p, g1, b1r, W2):
    def body(d0_ref, d1_ref, p_ref, g1_ref, b_ref, w_ref, o_ref):
        dinv = _dinv_of(d0_ref[...], d1_ref[...])
        h = (p_ref[0] + p_ref[1] - g1_ref[...]) * dinv + b_ref[...]
        h = jnp.maximum(h, 0.0)
        o_ref[...] = jnp.dot(h, w_ref[...],
                             preferred_element_type=jnp.float32) * dinv

    return pl.pallas_call(
        body,
        grid=_GRID,
        in_specs=[_DSPEC, _DSPEC,
                  pl.BlockSpec((NC, BM, D), lambda i: (0, i, 0)),
                  pl.BlockSpec((BM, D), lambda i: (i, 0)),
                  pl.BlockSpec((1, D), lambda i: (0, 0)),
                  pl.BlockSpec((D, D), lambda i: (0, 0))],
        out_specs=pl.BlockSpec((BM, D), lambda i: (i, 0)),
        out_shape=jax.ShapeDtypeStruct((N, D), jnp.float32),
    )(d0, d1, p, g1, b1r, W2)


def _post2(d0, d1, q, g2, b2r):
    def body(d0_ref, d1_ref, q_ref, g2_ref, b_ref, o_ref):
        dinv = _dinv_of(d0_ref[...], d1_ref[...])
        o_ref[...] = (q_ref[0] + q_ref[1] - g2_ref[...]) * dinv + b_ref[...]

    return pl.pallas_call(
        body,
        grid=_GRID,
        in_specs=[_DSPEC, _DSPEC,
                  pl.BlockSpec((NC, BM, D), lambda i: (0, i, 0)),
                  pl.BlockSpec((BM, D), lambda i: (i, 0)),
                  pl.BlockSpec((1, D), lambda i: (0, 0))],
        out_specs=pl.BlockSpec((BM, D), lambda i: (i, 0)),
        out_shape=jax.ShapeDtypeStruct((N, D), jnp.float32),
    )(d0, d1, q, g2, b2r)


def kernel(x, edge_index, W1, b1, W2, b2):
    ei = edge_index.astype(jnp.int32)
    pad = ((0, 0), (0, NCHP - NCH), (0, 0))
    src3 = jnp.pad(ei[0].reshape(NW, NCH, B), pad)
    dst3 = jnp.pad(ei[1].reshape(NW, NCH, B), pad)
    zeros1 = jnp.zeros((N,), jnp.float32)

    dp0, dp1 = _deg(dst3, zeros1)
    d0, d1 = dp0.reshape(N, 1), dp1.reshape(N, 1)
    g1 = _mmnorm(d0, d1, x, W1)
    p = _agg(g1, src3, dst3)
    g2 = _post1(d0, d1, p, g1, b1.reshape(1, D), W2)
    q = _agg(g2, src3, dst3)
    return _post2(d0, d1, q, g2, b2.reshape(1, D))


# speedup vs baseline: 36.7206x; 1.0158x over previous
import functools

import jax
import jax.numpy as jnp
from jax import lax
from jax.experimental import pallas as pl
from jax.experimental.pallas import tpu as pltpu
from jax.experimental.pallas import tpu_sc as plsc

N = 10000
D = 128
E = 320000
NC = 2
NS = 16
NW = NC * NS
EPW = E // NW
B = 80
NCH = EPW // B
DEG_W = 16

R0 = 624
R_LAST = N - (NS - 1) * R0

_MESH = plsc.VectorSubcoreMesh(core_axis_name="c", subcore_axis_name="s")


def _rows_copy(s, copy_fn):
    @pl.when(s < NS - 1)
    def _():
        copy_fn(s * R0, R0)

    @pl.when(s == NS - 1)
    def _():
        copy_fn((NS - 1) * R0, R_LAST)


def _deg_body(dst_hbm, zeros_hbm, out0_hbm, out1_hbm, acc, dst_v, ones_v,
              *sems):
    c = lax.axis_index("c")
    s = lax.axis_index("s")
    wid = c * NS + s

    @pl.when(s == 0)
    def _():
        pltpu.sync_copy(zeros_hbm, acc)

    pltpu.sync_copy(dst_hbm.at[wid], dst_v)

    @pl.loop(0, B // 16)
    def _(i):
        ones_v[pl.ds(i * 16, 16)] = jnp.ones((16,), jnp.float32)

    plsc.subcore_barrier()

    ndeep = len(sems)

    def fire(j, b):
        pltpu.async_copy(ones_v, acc.at[dst_v.at[j]], sems[b], add=True)

    def drain(b):
        pltpu.make_async_copy(ones_v, acc.at[dst_v.at[0]], sems[b]).wait()

    for b in range(ndeep):
        fire(b, b)

    @pl.loop(0, NCH // ndeep - 1)
    def _(t):
        j0 = (t + 1) * ndeep
        for b in range(ndeep):
            drain(b)
            fire(j0 + b, b)

    for r in range((NCH // ndeep) * ndeep, NCH):
        drain(r % ndeep)
        fire(r, r % ndeep)
    for r in range(NCH - ndeep, NCH):
        drain(r % ndeep)

    plsc.subcore_barrier()

    @pl.when(jnp.logical_and(c == 0, s == 0))
    def _():
        pltpu.sync_copy(acc, out0_hbm)

    @pl.when(jnp.logical_and(c == 1, s == 0))
    def _():
        pltpu.sync_copy(acc, out1_hbm)


_deg = functools.partial(
    pl.kernel,
    _deg_body,
    out_type=[jax.ShapeDtypeStruct((N,), jnp.float32),
              jax.ShapeDtypeStruct((N,), jnp.float32)],
    mesh=_MESH,
    scratch_types=[
        pltpu.VMEM_SHARED((N,), jnp.float32),
        pltpu.VMEM((NCH + 3, B), jnp.int32),
        pltpu.VMEM((B,), jnp.float32),
    ] + [pltpu.SemaphoreType.DMA] * 8,
)()


NGRP = 16
NFULL = NCH // NGRP
NTAIL = NCH - NFULL * NGRP
NCHP = (NFULL + 1) * NGRP
NBUF = 4


def _agg_body(g_hbm, src_hbm, dst_hbm, out_hbm, acc,
              sring0, sring1, dring0, dring1, buf0, buf1, buf2, buf3,
              ssem0, ssem1, dsem0, dsem1, gsem0, gsem1, gsem2, gsem3):
    srings, drings = (sring0, sring1), (dring0, dring1)
    ssems, dsems = (ssem0, ssem1), (dsem0, dsem1)
    bufs, gsems = (buf0, buf1, buf2, buf3), (gsem0, gsem1, gsem2, gsem3)
    c = lax.axis_index("c")
    s = lax.axis_index("s")
    wid = c * NS + s

    def stage(t, p):
        pltpu.async_copy(src_hbm.at[wid, pl.ds(t * NGRP, NGRP)],
                         srings[p], ssems[p])
        pltpu.async_copy(dst_hbm.at[wid, pl.ds(t * NGRP, NGRP)],
                         drings[p], dsems[p])

    def swait(p):
        pltpu.make_async_copy(src_hbm.at[wid, pl.ds(0, NGRP)],
                              srings[p], ssems[p]).wait()

    def dwait(p):
        pltpu.make_async_copy(dst_hbm.at[wid, pl.ds(0, NGRP)],
                              drings[p], dsems[p]).wait()

    def gfire(p, row, b):
        pltpu.async_copy(g_hbm.at[srings[p].at[row]], bufs[b], gsems[b])

    def gwait(b):
        pltpu.make_async_copy(g_hbm.at[srings[0].at[0]],
                              bufs[b], gsems[b]).wait()

    stage(0, 0)
    swait(0)
    stage(1, 1)
    for b in range(NBUF):
        gfire(0, b, b)
    _rows_copy(s, lambda off, sz: pltpu.sync_copy(
        g_hbm.at[pl.ds(off, sz)], acc.at[pl.ds(off, sz)]))
    plsc.subcore_barrier()

    def group_body(p, nch, has_next):
        dwait(p)
        for i in range(nch):
            b = i % NBUF
            gwait(b)
            pltpu.sync_copy(bufs[b], acc.at[drings[p].at[i]], add=True)
            if has_next:
                if i == NGRP - NBUF:
                    swait(1 - p)
                if i < NGRP - NBUF:
                    gfire(p, i + NBUF, b)
                else:
                    gfire(1 - p, i - (NGRP - NBUF), b)
            else:
                if i + NBUF < nch:
                    gfire(p, i + NBUF, b)

    group_body(0, NGRP, True)
    stage(2, 0)

    @pl.loop(0, (NFULL - 1) // 2)
    def _(u):
        t1 = 2 * u + 1
        group_body(1, NGRP, True)
        stage(t1 + 2, 1)
        group_body(0, NGRP, True)

        @pl.when(t1 + 3 <= NFULL)
        def _():
            stage(t1 + 3, 0)

    group_body(NFULL % 2, NTAIL, False)

    plsc.subcore_barrier()
    _rows_copy(s, lambda off, sz: pltpu.sync_copy(
        acc.at[pl.ds(off, sz)], out_hbm.at[c, pl.ds(off, sz)]))


_agg = functools.partial(
    pl.kernel,
    _agg_body,
    out_type=jax.ShapeDtypeStruct((NC, N, D), jnp.float32),
    mesh=_MESH,
    scratch_types=[
        pltpu.VMEM_SHARED((N, D), jnp.float32),
        pltpu.VMEM((NGRP, B), jnp.int32),
        pltpu.VMEM((NGRP, B), jnp.int32),
        pltpu.VMEM((NGRP, B), jnp.int32),
        pltpu.VMEM((NGRP, B), jnp.int32),
        pltpu.VMEM((B, D), jnp.float32),
        pltpu.VMEM((B, D), jnp.float32),
        pltpu.VMEM((B, D), jnp.float32),
        pltpu.VMEM((B, D), jnp.float32),
    ] + [pltpu.SemaphoreType.DMA] * 8,
)()


BM = 1000
_GRID = (pl.cdiv(N, BM),)


_DSPEC = pl.BlockSpec((BM, 1), lambda i: (i, 0))


def _dinv_of(d0, d1):
    return lax.rsqrt(d0 + d1 + 1.0)


def _mmnorm(d0, d1, x, W):
    def body(d0_ref, d1_ref, x_ref, w_ref, o_ref):
        o_ref[...] = jnp.dot(x_ref[...], w_ref[...],
                             preferred_element_type=jnp.float32
                             ) * _dinv_of(d0_ref[...], d1_ref[...])

    return pl.pallas_call(
        body,
        grid=_GRID,
        in_specs=[_DSPEC, _DSPEC,
                  pl.BlockSpec((BM, D), lambda i: (i, 0)),
                  pl.BlockSpec((D, D), lambda i: (0, 0))],
        out_specs=pl.BlockSpec((BM, D), lambda i: (i, 0)),
        out_shape=jax.ShapeDtypeStruct((N, D), jnp.float32),
    )(d0, d1, x, W)


def _post1(d0, d1, p, g1, b1r, W2):
    def body(d0_ref, d1_ref, p_ref, g1_ref, b_ref, w_ref, o_ref):
        dinv = _dinv_of(d0_ref[...], d1_ref[...])
        h = (p_ref[0] + p_ref[1] - g1_ref[...]) * dinv + b_ref[...]
        h = jnp.maximum(h, 0.0)
        o_ref[...] = jnp.dot(h, w_ref[...],
                             preferred_element_type=jnp.float32) * dinv

    return pl.pallas_call(
        body,
        grid=_GRID,
        in_specs=[_DSPEC, _DSPEC,
                  pl.BlockSpec((NC, BM, D), lambda i: (0, i, 0)),
                  pl.BlockSpec((BM, D), lambda i: (i, 0)),
                  pl.BlockSpec((1, D), lambda i: (0, 0)),
                  pl.BlockSpec((D, D), lambda i: (0, 0))],
        out_specs=pl.BlockSpec((BM, D), lambda i: (i, 0)),
        out_shape=jax.ShapeDtypeStruct((N, D), jnp.float32),
    )(d0, d1, p, g1, b1r, W2)


def _post2(d0, d1, q, g2, b2r):
    def body(d0_ref, d1_ref, q_ref, g2_ref, b_ref, o_ref):
        dinv = _dinv_of(d0_ref[...], d1_ref[...])
        o_ref[...] = (q_ref[0] + q_ref[1] - g2_ref[...]) * dinv + b_ref[...]

    return pl.pallas_call(
        body,
        grid=_GRID,
        in_specs=[_DSPEC, _DSPEC,
                  pl.BlockSpec((NC, BM, D), lambda i: (0, i, 0)),
                  pl.BlockSpec((BM, D), lambda i: (i, 0)),
                  pl.BlockSpec((1, D), lambda i: (0, 0))],
        out_specs=pl.BlockSpec((BM, D), lambda i: (i, 0)),
        out_shape=jax.ShapeDtypeStruct((N, D), jnp.float32),
    )(d0, d1, q, g2, b2r)


def kernel(x, edge_index, W1, b1, W2, b2):
    ei = edge_index.astype(jnp.int32)
    pad = ((0, 0), (0, NCHP - NCH), (0, 0))
    src3 = jnp.pad(ei[0].reshape(NW, NCH, B), pad)
    dst3 = jnp.pad(ei[1].reshape(NW, NCH, B), pad)
    zeros1 = jnp.zeros((N,), jnp.float32)

    dp0, dp1 = _deg(dst3, zeros1)
    d0, d1 = dp0.reshape(N, 1), dp1.reshape(N, 1)
    g1 = _mmnorm(d0, d1, x, W1)
    p = _agg(g1, src3, dst3)
    g2 = _post1(d0, d1, p, g1, b1.reshape(1, D), W2)
    q = _agg(g2, src3, dst3)
    return _post2(d0, d1, q, g2, b2.reshape(1, D))


# speedup vs baseline: 38.1114x; 1.0379x over previous
import functools

import jax
import jax.numpy as jnp
from jax import lax
from jax.experimental import pallas as pl
from jax.experimental.pallas import tpu as pltpu
from jax.experimental.pallas import tpu_sc as plsc

N = 10000
D = 128
E = 320000
NC = 2
NS = 16
NW = NC * NS
EPW = E // NW
B = 80
NCH = EPW // B
DEG_W = 16

R0 = 624
R_LAST = N - (NS - 1) * R0

_MESH = plsc.VectorSubcoreMesh(core_axis_name="c", subcore_axis_name="s")


def _rows_copy(s, copy_fn):
    @pl.when(s < NS - 1)
    def _():
        copy_fn(s * R0, R0)

    @pl.when(s == NS - 1)
    def _():
        copy_fn((NS - 1) * R0, R_LAST)


def _deg_body(e4_hbm, zeros_hbm, out0_hbm, out1_hbm, acc, dst_v, ones_v,
              *sems):
    c = lax.axis_index("c")
    s = lax.axis_index("s")
    wid = c * NS + s

    @pl.when(s == 0)
    def _():
        pltpu.sync_copy(zeros_hbm, acc)

    pltpu.sync_copy(e4_hbm.at[1, wid], dst_v)

    @pl.loop(0, B // 16)
    def _(i):
        ones_v[pl.ds(i * 16, 16)] = jnp.ones((16,), jnp.float32)

    plsc.subcore_barrier()

    ndeep = len(sems)

    def fire(j, b):
        pltpu.async_copy(ones_v, acc.at[dst_v.at[j]], sems[b], add=True)

    def drain(b):
        pltpu.make_async_copy(ones_v, acc.at[dst_v.at[0]], sems[b]).wait()

    for b in range(ndeep):
        fire(b, b)

    @pl.loop(0, NCH // ndeep - 1)
    def _(t):
        j0 = (t + 1) * ndeep
        for b in range(ndeep):
            drain(b)
            fire(j0 + b, b)

    for r in range((NCH // ndeep) * ndeep, NCH):
        drain(r % ndeep)
        fire(r, r % ndeep)
    for r in range(NCH - ndeep, NCH):
        drain(r % ndeep)

    plsc.subcore_barrier()

    @pl.when(jnp.logical_and(c == 0, s == 0))
    def _():
        pltpu.sync_copy(acc, out0_hbm)

    @pl.when(jnp.logical_and(c == 1, s == 0))
    def _():
        pltpu.sync_copy(acc, out1_hbm)


_deg = functools.partial(
    pl.kernel,
    _deg_body,
    out_type=[jax.ShapeDtypeStruct((N,), jnp.float32),
              jax.ShapeDtypeStruct((N,), jnp.float32)],
    mesh=_MESH,
    scratch_types=[
        pltpu.VMEM_SHARED((N,), jnp.float32),
        pltpu.VMEM((NCH + 3, B), jnp.int32),
        pltpu.VMEM((B,), jnp.float32),
    ] + [pltpu.SemaphoreType.DMA] * 8,
)()


NGRP = 16
NFULL = NCH // NGRP
NTAIL = NCH - NFULL * NGRP
NCHP = (NFULL + 1) * NGRP
NBUF = 4


def _agg_body(g_hbm, e4_hbm, out_hbm, acc,
              sring0, sring1, dring0, dring1, buf0, buf1, buf2, buf3,
              ssem0, ssem1, dsem0, dsem1, gsem0, gsem1, gsem2, gsem3):
    srings, drings = (sring0, sring1), (dring0, dring1)
    ssems, dsems = (ssem0, ssem1), (dsem0, dsem1)
    bufs, gsems = (buf0, buf1, buf2, buf3), (gsem0, gsem1, gsem2, gsem3)
    c = lax.axis_index("c")
    s = lax.axis_index("s")
    wid = c * NS + s

    def stage(t, p):
        pltpu.async_copy(e4_hbm.at[0, wid, pl.ds(t * NGRP, NGRP)],
                         srings[p], ssems[p])
        pltpu.async_copy(e4_hbm.at[1, wid, pl.ds(t * NGRP, NGRP)],
                         drings[p], dsems[p])

    def swait(p):
        pltpu.make_async_copy(e4_hbm.at[0, wid, pl.ds(0, NGRP)],
                              srings[p], ssems[p]).wait()

    def dwait(p):
        pltpu.make_async_copy(e4_hbm.at[1, wid, pl.ds(0, NGRP)],
                              drings[p], dsems[p]).wait()

    def gfire(p, row, b):
        pltpu.async_copy(g_hbm.at[srings[p].at[row]], bufs[b], gsems[b])

    def gwait(b):
        pltpu.make_async_copy(g_hbm.at[srings[0].at[0]],
                              bufs[b], gsems[b]).wait()

    stage(0, 0)
    swait(0)
    stage(1, 1)
    for b in range(NBUF):
        gfire(0, b, b)
    _rows_copy(s, lambda off, sz: pltpu.sync_copy(
        g_hbm.at[pl.ds(off, sz)], acc.at[pl.ds(off, sz)]))
    plsc.subcore_barrier()

    def group_body(p, nch, has_next):
        dwait(p)
        for i in range(nch):
            b = i % NBUF
            gwait(b)
            pltpu.sync_copy(bufs[b], acc.at[drings[p].at[i]], add=True)
            if has_next:
                if i == NGRP - NBUF:
                    swait(1 - p)
                if i < NGRP - NBUF:
                    gfire(p, i + NBUF, b)
                else:
                    gfire(1 - p, i - (NGRP - NBUF), b)
            else:
                if i + NBUF < nch:
                    gfire(p, i + NBUF, b)

    group_body(0, NGRP, True)
    stage(2, 0)

    @pl.loop(0, (NFULL - 1) // 2)
    def _(u):
        t1 = 2 * u + 1
        group_body(1, NGRP, True)
        stage(t1 + 2, 1)
        group_body(0, NGRP, True)

        @pl.when(t1 + 3 <= NFULL)
        def _():
            stage(t1 + 3, 0)

    group_body(NFULL % 2, NTAIL, False)

    plsc.subcore_barrier()
    _rows_copy(s, lambda off, sz: pltpu.sync_copy(
        acc.at[pl.ds(off, sz)], out_hbm.at[c, pl.ds(off, sz)]))


_agg = functools.partial(
    pl.kernel,
    _agg_body,
    out_type=jax.ShapeDtypeStruct((NC, N, D), jnp.float32),
    mesh=_MESH,
    scratch_types=[
        pltpu.VMEM_SHARED((N, D), jnp.float32),
        pltpu.VMEM((NGRP, B), jnp.int32),
        pltpu.VMEM((NGRP, B), jnp.int32),
        pltpu.VMEM((NGRP, B), jnp.int32),
        pltpu.VMEM((NGRP, B), jnp.int32),
        pltpu.VMEM((B, D), jnp.float32),
        pltpu.VMEM((B, D), jnp.float32),
        pltpu.VMEM((B, D), jnp.float32),
        pltpu.VMEM((B, D), jnp.float32),
    ] + [pltpu.SemaphoreType.DMA] * 8,
)()


BM = 1000
_GRID = (pl.cdiv(N, BM),)


_DSPEC = pl.BlockSpec((BM, 1), lambda i: (i, 0))


def _dinv_of(d0, d1):
    return lax.rsqrt(d0 + d1 + 1.0)


def _mmnorm(d0, d1, x, W):
    def body(d0_ref, d1_ref, x_ref, w_ref, o_ref):
        o_ref[...] = jnp.dot(x_ref[...], w_ref[...],
                             preferred_element_type=jnp.float32
                             ) * _dinv_of(d0_ref[...], d1_ref[...])

    return pl.pallas_call(
        body,
        grid=_GRID,
        in_specs=[_DSPEC, _DSPEC,
                  pl.BlockSpec((BM, D), lambda i: (i, 0)),
                  pl.BlockSpec((D, D), lambda i: (0, 0))],
        out_specs=pl.BlockSpec((BM, D), lambda i: (i, 0)),
        out_shape=jax.ShapeDtypeStruct((N, D), jnp.float32),
    )(d0, d1, x, W)


def _post1(d0, d1, p, g1, b1r, W2):
    def body(d0_ref, d1_ref, p_ref, g1_ref, b_ref, w_ref, o_ref):
        dinv = _dinv_of(d0_ref[...], d1_ref[...])
        h = (p_ref[0] + p_ref[1] - g1_ref[...]) * dinv + b_ref[...]
        h = jnp.maximum(h, 0.0)
        o_ref[...] = jnp.dot(h, w_ref[...],
                             preferred_element_type=jnp.float32) * dinv

    return pl.pallas_call(
        body,
        grid=_GRID,
        in_specs=[_DSPEC, _DSPEC,
                  pl.BlockSpec((NC, BM, D), lambda i: (0, i, 0)),
                  pl.BlockSpec((BM, D), lambda i: (i, 0)),
                  pl.BlockSpec((1, D), lambda i: (0, 0)),
                  pl.BlockSpec((D, D), lambda i: (0, 0))],
        out_specs=pl.BlockSpec((BM, D), lambda i: (i, 0)),
        out_shape=jax.ShapeDtypeStruct((N, D), jnp.float32),
    )(d0, d1, p, g1, b1r, W2)


def _post2(d0, d1, q, g2, b2r):
    def body(d0_ref, d1_ref, q_ref, g2_ref, b_ref, o_ref):
        dinv = _dinv_of(d0_ref[...], d1_ref[...])
        o_ref[...] = (q_ref[0] + q_ref[1] - g2_ref[...]) * dinv + b_ref[...]

    return pl.pallas_call(
        body,
        grid=_GRID,
        in_specs=[_DSPEC, _DSPEC,
                  pl.BlockSpec((NC, BM, D), lambda i: (0, i, 0)),
                  pl.BlockSpec((BM, D), lambda i: (i, 0)),
                  pl.BlockSpec((1, D), lambda i: (0, 0))],
        out_specs=pl.BlockSpec((BM, D), lambda i: (i, 0)),
        out_shape=jax.ShapeDtypeStruct((N, D), jnp.float32),
    )(d0, d1, q, g2, b2r)


def kernel(x, edge_index, W1, b1, W2, b2):
    e4 = jnp.pad(edge_index.astype(jnp.int32).reshape(2, NW, NCH, B),
                 ((0, 0), (0, 0), (0, NCHP - NCH), (0, 0)))
    zeros1 = jnp.zeros((N,), jnp.float32)

    dp0, dp1 = _deg(e4, zeros1)
    d0, d1 = dp0.reshape(N, 1), dp1.reshape(N, 1)
    g1 = _mmnorm(d0, d1, x, W1)
    p = _agg(g1, e4)
    g2 = _post1(d0, d1, p, g1, b1.reshape(1, D), W2)
    q = _agg(g2, e4)
    return _post2(d0, d1, q, g2, b2.reshape(1, D))
